# Initial kernel scaffold; baseline (speedup 1.0000x reference)
import jax
import jax.numpy as jnp
from jax.experimental import pallas as pl


def kernel(class_probits, pred_boxes, pred_labels, gt):
    raise NotImplementedError("write your pallas kernel here")


# speedup vs baseline: 578.6119x; 578.6119x over previous
import jax
import jax.numpy as jnp
from jax.experimental import pallas as pl
from jax.experimental.pallas import tpu as pltpu

_EPS = 1e-05
_THR = 0.5
_NCLS = 4
_BPC = 512
_NB = (_NCLS - 1) * _BPC
_SENT = 2 ** 20
_BIG = 2 ** 30
_TILE = 512
_RT = 256


def _bucket_body(px0, py0, px1, py1, plab, gx0, gy0, gx1, gy1, glab, bout):
    a2 = (px1[...] - px0[...]) * (py1[...] - py0[...])
    a1 = (gx1[...] - gx0[...]) * (gy1[...] - gy0[...])
    ltx = jnp.maximum(gx0[...], px0[...])
    lty = jnp.maximum(gy0[...], py0[...])
    rbx = jnp.minimum(gx1[...], px1[...])
    rby = jnp.minimum(gy1[...], py1[...])
    wx = jnp.clip(rbx - ltx, 0.0, None)
    wy = jnp.clip(rby - lty, 0.0, None)
    inter = wx * wy
    union = a1 + a2 - inter
    iou = inter / union
    match = glab[...] == plab[...].astype(jnp.float32)
    iou_m = jnp.where(match, iou, -jnp.inf)
    m = jnp.max(iou_m, axis=0, keepdims=True)
    anym = m > _THR
    gp = iou_m.shape[0]
    rowi = jax.lax.broadcasted_iota(jnp.int32, iou_m.shape, 0)
    gstar = jnp.min(jnp.where(iou_m == m, rowi, gp), axis=0, keepdims=True)
    j = jnp.sum(((iou_m > _THR) & (rowi < gstar)).astype(jnp.int32),
                axis=0, keepdims=True)
    bout[...] = jnp.where(anym, (plab[...] - 1) * _BPC + j,
                          jnp.int32(_SENT))


def _final_body(prob, lab, buck, glab, out):
    npad = prob.shape[0]
    nrt = npad // _RT
    lanes = jax.lax.broadcasted_iota(jnp.int32, (1, _NB), 1)
    cls_lane = 1 + (lanes // _BPC)

    def p1(i, acc):
        pb = prob[pl.ds(i * _RT, _RT), :]
        bk = buck[pl.ds(i * _RT, _RT), :]
        hit = bk == lanes
        return jnp.maximum(
            acc, jnp.max(jnp.where(hit, pb, -1.0), axis=0, keepdims=True))

    bmax = jax.lax.fori_loop(0, nrt, p1, jnp.full((1, _NB), -1.0, jnp.float32))

    def p2(i, acc):
        pb = prob[pl.ds(i * _RT, _RT), :]
        bk = buck[pl.ds(i * _RT, _RT), :]
        ridx = jax.lax.broadcasted_iota(jnp.int32, (_RT, 1), 0) + i * _RT
        hit = (bk == lanes) & (pb == bmax)
        return jnp.minimum(
            acc, jnp.min(jnp.where(hit, ridx, _BIG), axis=0, keepdims=True))

    bidx = jax.lax.fori_loop(0, nrt, p2, jnp.full((1, _NB), _BIG, jnp.int32))

    def p3(i, carry):
        acc, np1, np2, np3 = carry
        pb = prob[pl.ds(i * _RT, _RT), :]
        lb = lab[pl.ds(i * _RT, _RT), :]
        ridx = jax.lax.broadcasted_iota(jnp.int32, (_RT, 1), 0) + i * _RT
        clsm = lb == cls_lane
        ahead = (pb > bmax) | ((pb == bmax) & (ridx < bidx))
        acc = acc + jnp.sum((clsm & ahead).astype(jnp.int32),
                            axis=0, keepdims=True)
        np1 = np1 + jnp.sum((lb == 1).astype(jnp.int32))
        np2 = np2 + jnp.sum((lb == 2).astype(jnp.int32))
        np3 = np3 + jnp.sum((lb == 3).astype(jnp.int32))
        return acc, np1, np2, np3

    zero = jnp.int32(0)
    rk, np1, np2, np3 = jax.lax.fori_loop(
        0, nrt, p3, (jnp.zeros((1, _NB), jnp.int32), zero, zero, zero))
    rk = jnp.where(bidx == _BIG, _BIG, rk)

    g = glab[...]
    ng1 = jnp.sum((g == 1.0).astype(jnp.float32))
    ng2 = jnp.sum((g == 2.0).astype(jnp.float32))
    ng3 = jnp.sum((g == 3.0).astype(jnp.float32))

    def p4(i, carry):
        s1, s2, s3 = carry
        i0 = jax.lax.broadcasted_iota(jnp.int32, (_RT, 1), 0) + i * _RT

        def one(c, ngc, npc):
            t = jnp.sum(((rk <= i0) & (cls_lane == c)).astype(jnp.float32),
                        axis=1, keepdims=True)
            fpc = (i0 + 1).astype(jnp.float32) - t
            prec = t / (t + fpc + _EPS)
            rec = fpc / (ngc + _EPS)
            ratio = jnp.where(i0 < npc, prec / rec, 0.0)
            return jnp.sum(ratio)

        s1 = s1 + one(1, ng1, np1)
        s2 = s2 + one(2, ng2, np2)
        s3 = s3 + one(3, ng3, np3)
        return s1, s2, s3

    zf = jnp.float32(0.0)
    s1, s2, s3 = jax.lax.fori_loop(0, nrt, p4, (zf, zf, zf))

    def ap_of(s, npc, ngc):
        empty = jnp.logical_or(npc == 0, ngc == 0.0)
        return jnp.where(empty, 0.0, s / npc.astype(jnp.float32))

    a1 = ap_of(s1, np1, ng1)
    a2 = ap_of(s2, np2, ng2)
    a3 = ap_of(s3, np3, ng3)
    out[...] = ((a1 + a2 + a3) / 3.0) + jnp.zeros((1, 1), jnp.float32)


def kernel(class_probits, pred_boxes, pred_labels, gt):
    n = pred_boxes.shape[0]
    gt2 = jnp.squeeze(gt, axis=0)
    g = gt2.shape[0]
    npad = ((n + _TILE - 1) // _TILE) * _TILE
    gpad = ((g + 511) // 512) * 512

    def prow(x):
        return jnp.pad(x, (0, npad - n)).reshape(1, npad)

    def gcol(x, fill=0.0):
        return jnp.pad(x, (0, gpad - g), constant_values=fill).reshape(gpad, 1)

    px0 = prow(pred_boxes[:, 0])
    py0 = prow(pred_boxes[:, 1])
    px1 = prow(pred_boxes[:, 2])
    py1 = prow(pred_boxes[:, 3])
    plab = prow(pred_labels.astype(jnp.int32))
    gx0 = gcol(gt2[:, 1])
    gy0 = gcol(gt2[:, 2])
    gx1 = gcol(gt2[:, 3])
    gy1 = gcol(gt2[:, 4])
    glab = gcol(gt2[:, 0], fill=-1.0)

    grid = npad // _TILE
    pspec = pl.BlockSpec((1, _TILE), lambda i: (0, i))
    gspec = pl.BlockSpec((gpad, 1), lambda i: (0, 0))
    bucket = pl.pallas_call(
        _bucket_body,
        grid=(grid,),
        in_specs=[pspec] * 5 + [gspec] * 5,
        out_specs=pspec,
        out_shape=jax.ShapeDtypeStruct((1, npad), jnp.int32),
    )(px0, py0, px1, py1, plab, gx0, gy0, gx1, gy1, glab)

    prob_col = jnp.pad(class_probits, (0, npad - n)).reshape(npad, 1)
    lab_col = plab.reshape(npad, 1)
    buck_col = bucket.reshape(npad, 1)

    out = pl.pallas_call(
        _final_body,
        in_specs=[pl.BlockSpec(x.shape, lambda: (0,) * x.ndim)
                  for x in (prob_col, lab_col, buck_col, glab)],
        out_specs=pl.BlockSpec((1, 1), lambda: (0, 0)),
        out_shape=jax.ShapeDtypeStruct((1, 1), jnp.float32),
    )(prob_col, lab_col, buck_col, glab)
    return out[0, 0]
